# Initial kernel scaffold; baseline (speedup 1.0000x reference)
import jax
import jax.numpy as jnp
from jax.experimental import pallas as pl


def kernel(xyz, skip, xyz_prev, feat_prev, W1, g1, b1, W2, g2, b2):
    raise NotImplementedError("write your pallas kernel here")


# speedup vs baseline: 20.7253x; 20.7253x over previous
import functools

import jax
import jax.numpy as jnp
from jax.experimental import pallas as pl

_INTERPRET = False
_PREC = jax.lax.Precision.HIGHEST


def _three_nn_kernel(q_ref, p_ref, ind_ref, w_ref, *, n_keys):
    q = q_ref[0]
    p = p_ref[0]
    pp = jnp.sum(p * p, axis=1, keepdims=True)
    qp = jax.lax.dot_general(p, q, (((1,), (0,)), ((), ())),
                             precision=jax.lax.Precision.DEFAULT)
    s = pp - 2.0 * qp
    qq = jnp.sum(q * q, axis=0, keepdims=True)
    iota = jax.lax.broadcasted_iota(jnp.int32, s.shape, 0)
    mins, args = [], []
    cur = s
    for k in range(3):
        m = jnp.min(cur, axis=0, keepdims=True)
        a = jnp.min(jnp.where(cur == m, iota, n_keys),
                    axis=0, keepdims=True)
        mins.append(m)
        args.append(a)
        if k < 2:
            cur = jnp.where(iota == a, jnp.inf, cur)
    sqs = [jnp.maximum(m + qq, 0.0) for m in mins]
    dists = [jnp.where(d < 1e-10, 1e-10, d) for d in sqs]
    invs = [1.0 / (d + 1e-8) for d in dists]
    norm = invs[0] + invs[1] + invs[2]
    ind_ref[0] = jnp.concatenate(args, axis=0)
    w_ref[0] = jnp.concatenate([iv / norm for iv in invs], axis=0)


def _interp_conv1_kernel(ind_ref, w_ref, fp_ref, skip_ref, W1_ref,
                         y1_ref, st_ref, *, n_keys, cprev):
    b = pl.program_id(0)
    t = pl.program_id(1)

    @pl.when(jnp.logical_and(b == 0, t == 0))
    def _init():
        st_ref[...] = jnp.zeros_like(st_ref)

    ind = ind_ref[0]
    w = w_ref[0]
    fp = fp_ref[0]
    tq = ind.shape[1]
    iota = jax.lax.broadcasted_iota(jnp.int32, (n_keys, tq), 0)
    S = jnp.zeros((n_keys, tq), jnp.float32)
    for k in range(3):
        S = S + jnp.where(iota == ind[k:k + 1, :], w[k:k + 1, :], 0.0)
    feats = jax.lax.dot_general(fp, S, (((1,), (0,)), ((), ())),
                                precision=_PREC)
    skipb = skip_ref[0]
    W1 = W1_ref[...]
    y1 = (jax.lax.dot_general(W1[:, :cprev], feats,
                              (((1,), (0,)), ((), ())), precision=_PREC)
          + jax.lax.dot_general(W1[:, cprev:], skipb,
                                (((1,), (0,)), ((), ())), precision=_PREC))
    y1_ref[0] = y1
    s1 = jnp.sum(y1, axis=1, keepdims=True)
    s2 = jnp.sum(y1 * y1, axis=1, keepdims=True)
    st_ref[...] += jnp.concatenate([s1, s2], axis=1)


def _bn_conv2_kernel(y1_ref, st_ref, g_ref, b_ref, W2_ref,
                     y2_ref, st2_ref, *, count):
    b = pl.program_id(0)
    t = pl.program_id(1)

    @pl.when(jnp.logical_and(b == 0, t == 0))
    def _init():
        st2_ref[...] = jnp.zeros_like(st2_ref)

    st = st_ref[...]
    inv_cnt = 1.0 / count
    mean = st[:, 0:1] * inv_cnt
    var = st[:, 1:2] * inv_cnt - mean * mean
    inv = jax.lax.rsqrt(var + 1e-3)
    z = (y1_ref[0] - mean) * inv * g_ref[...] + b_ref[...]
    z = jnp.where(z >= 0, z, 0.01 * z)
    y2 = jax.lax.dot_general(W2_ref[...], z, (((1,), (0,)), ((), ())),
                             precision=_PREC)
    y2_ref[0] = y2
    s1 = jnp.sum(y2, axis=1, keepdims=True)
    s2 = jnp.sum(y2 * y2, axis=1, keepdims=True)
    st2_ref[...] += jnp.concatenate([s1, s2], axis=1)


def _bn_out_kernel(y2_ref, st_ref, g_ref, b_ref, out_ref, *, count):
    st = st_ref[...]
    inv_cnt = 1.0 / count
    mean = st[:, 0:1] * inv_cnt
    var = st[:, 1:2] * inv_cnt - mean * mean
    inv = jax.lax.rsqrt(var + 1e-3)
    z = (y2_ref[0] - mean) * inv * g_ref[...] + b_ref[...]
    out_ref[0] = jnp.where(z >= 0, z, 0.01 * z)


def kernel(xyz, skip, xyz_prev, feat_prev, W1, g1, b1, W2, g2, b2):
    B, _, N0 = xyz.shape
    N = xyz_prev.shape[2]
    Cprev = feat_prev.shape[1]
    Cskip = skip.shape[1]
    C1 = W1.shape[0]
    C2 = W2.shape[0]
    TQ = 512
    nt = N0 // TQ
    count = float(B * N0)

    p_t = jnp.transpose(xyz_prev, (0, 2, 1))

    ind_t, w_t = pl.pallas_call(
        functools.partial(_three_nn_kernel, n_keys=N),
        grid=(B, nt),
        in_specs=[pl.BlockSpec((1, 3, TQ), lambda b, t: (b, 0, t)),
                  pl.BlockSpec((1, N, 3), lambda b, t: (b, 0, 0))],
        out_specs=[pl.BlockSpec((1, 3, TQ), lambda b, t: (b, 0, t)),
                   pl.BlockSpec((1, 3, TQ), lambda b, t: (b, 0, t))],
        out_shape=[jax.ShapeDtypeStruct((B, 3, N0), jnp.int32),
                   jax.ShapeDtypeStruct((B, 3, N0), jnp.float32)],
        interpret=_INTERPRET,
    )(xyz, p_t)

    y1, st1 = pl.pallas_call(
        functools.partial(_interp_conv1_kernel, n_keys=N, cprev=Cprev),
        grid=(B, nt),
        in_specs=[pl.BlockSpec((1, 3, TQ), lambda b, t: (b, 0, t)),
                  pl.BlockSpec((1, 3, TQ), lambda b, t: (b, 0, t)),
                  pl.BlockSpec((1, Cprev, N), lambda b, t: (b, 0, 0)),
                  pl.BlockSpec((1, Cskip, TQ), lambda b, t: (b, 0, t)),
                  pl.BlockSpec((C1, Cprev + Cskip), lambda b, t: (0, 0))],
        out_specs=[pl.BlockSpec((1, C1, TQ), lambda b, t: (b, 0, t)),
                   pl.BlockSpec((C1, 2), lambda b, t: (0, 0))],
        out_shape=[jax.ShapeDtypeStruct((B, C1, N0), jnp.float32),
                   jax.ShapeDtypeStruct((C1, 2), jnp.float32)],
        interpret=_INTERPRET,
    )(ind_t, w_t, feat_prev, skip, W1)

    y2, st2 = pl.pallas_call(
        functools.partial(_bn_conv2_kernel, count=count),
        grid=(B, nt),
        in_specs=[pl.BlockSpec((1, C1, TQ), lambda b, t: (b, 0, t)),
                  pl.BlockSpec((C1, 2), lambda b, t: (0, 0)),
                  pl.BlockSpec((C1, 1), lambda b, t: (0, 0)),
                  pl.BlockSpec((C1, 1), lambda b, t: (0, 0)),
                  pl.BlockSpec((C2, C1), lambda b, t: (0, 0))],
        out_specs=[pl.BlockSpec((1, C2, TQ), lambda b, t: (b, 0, t)),
                   pl.BlockSpec((C2, 2), lambda b, t: (0, 0))],
        out_shape=[jax.ShapeDtypeStruct((B, C2, N0), jnp.float32),
                   jax.ShapeDtypeStruct((C2, 2), jnp.float32)],
        interpret=_INTERPRET,
    )(y1, st1, g1.reshape(-1, 1), b1.reshape(-1, 1), W2)

    y = pl.pallas_call(
        functools.partial(_bn_out_kernel, count=count),
        grid=(B, nt),
        in_specs=[pl.BlockSpec((1, C2, TQ), lambda b, t: (b, 0, t)),
                  pl.BlockSpec((C2, 2), lambda b, t: (0, 0)),
                  pl.BlockSpec((C2, 1), lambda b, t: (0, 0)),
                  pl.BlockSpec((C2, 1), lambda b, t: (0, 0))],
        out_specs=pl.BlockSpec((1, C2, TQ), lambda b, t: (b, 0, t)),
        out_shape=jax.ShapeDtypeStruct((B, C2, N0), jnp.float32),
        interpret=_INTERPRET,
    )(y2, st2, g2.reshape(-1, 1), b2.reshape(-1, 1))

    return (xyz, y)


# speedup vs baseline: 30.1228x; 1.4534x over previous
import functools

import jax
import jax.numpy as jnp
from jax.experimental import pallas as pl

_INTERPRET = False
_PREC = jax.lax.Precision.DEFAULT


def _three_nn_kernel(q_ref, p_ref, ind_ref, w_ref, *, n_keys):
    q = q_ref[0]
    p = p_ref[0]
    pp = jnp.sum(p * p, axis=1, keepdims=True)
    qp = jax.lax.dot_general(p, q, (((1,), (0,)), ((), ())),
                             precision=jax.lax.Precision.DEFAULT)
    s = pp - 2.0 * qp
    qq = jnp.sum(q * q, axis=0, keepdims=True)
    iota = jax.lax.broadcasted_iota(jnp.int32, s.shape, 0)
    mins, args = [], []
    cur = s
    for k in range(3):
        m = jnp.min(cur, axis=0, keepdims=True)
        a = jnp.min(jnp.where(cur == m, iota, n_keys),
                    axis=0, keepdims=True)
        mins.append(m)
        args.append(a)
        if k < 2:
            cur = jnp.where(iota == a, jnp.inf, cur)
    sqs = [jnp.maximum(m + qq, 0.0) for m in mins]
    dists = [jnp.where(d < 1e-10, 1e-10, d) for d in sqs]
    invs = [1.0 / (d + 1e-8) for d in dists]
    norm = invs[0] + invs[1] + invs[2]
    ind_ref[0] = jnp.concatenate(args, axis=0)
    w_ref[0] = jnp.concatenate([iv / norm for iv in invs], axis=0)


def _interp_conv1_kernel(ind_ref, w_ref, fp_ref, skip_ref, W1_ref,
                         y1_ref, st_ref, *, n_keys, cprev):
    b = pl.program_id(0)
    t = pl.program_id(1)

    @pl.when(jnp.logical_and(b == 0, t == 0))
    def _init():
        st_ref[...] = jnp.zeros_like(st_ref)

    ind = ind_ref[0]
    w = w_ref[0]
    fp = fp_ref[0]
    tq = ind.shape[1]
    iota = jax.lax.broadcasted_iota(jnp.int32, (n_keys, tq), 0)
    sels = [jnp.where(iota == ind[k:k + 1, :], w[k:k + 1, :], 0.0)
            for k in range(3)]
    S = (sels[0] + sels[1]) + sels[2]
    feats = jax.lax.dot_general(fp, S, (((1,), (0,)), ((), ())),
                                precision=_PREC)
    skipb = skip_ref[0]
    W1 = W1_ref[...]
    y1 = (jax.lax.dot_general(W1[:, :cprev], feats,
                              (((1,), (0,)), ((), ())), precision=_PREC)
          + jax.lax.dot_general(W1[:, cprev:], skipb,
                                (((1,), (0,)), ((), ())), precision=_PREC))
    y1_ref[0] = y1
    s1 = jnp.sum(y1, axis=1, keepdims=True)
    s2 = jnp.sum(y1 * y1, axis=1, keepdims=True)
    st_ref[...] += jnp.concatenate([s1, s2], axis=1)


def _bn_conv2_kernel(y1_ref, st_ref, g_ref, b_ref, W2_ref,
                     y2_ref, st2_ref, *, count):
    b = pl.program_id(0)
    t = pl.program_id(1)

    @pl.when(jnp.logical_and(b == 0, t == 0))
    def _init():
        st2_ref[...] = jnp.zeros_like(st2_ref)

    st = st_ref[...]
    inv_cnt = 1.0 / count
    mean = st[:, 0:1] * inv_cnt
    var = st[:, 1:2] * inv_cnt - mean * mean
    inv = jax.lax.rsqrt(var + 1e-3)
    z = (y1_ref[0] - mean) * inv * g_ref[...] + b_ref[...]
    z = jnp.where(z >= 0, z, 0.01 * z)
    y2 = jax.lax.dot_general(W2_ref[...], z, (((1,), (0,)), ((), ())),
                             precision=_PREC)
    y2_ref[0] = y2
    s1 = jnp.sum(y2, axis=1, keepdims=True)
    s2 = jnp.sum(y2 * y2, axis=1, keepdims=True)
    st2_ref[...] += jnp.concatenate([s1, s2], axis=1)


def _bn_out_kernel(y2_ref, st_ref, g_ref, b_ref, out_ref, *, count):
    st = st_ref[...]
    inv_cnt = 1.0 / count
    mean = st[:, 0:1] * inv_cnt
    var = st[:, 1:2] * inv_cnt - mean * mean
    inv = jax.lax.rsqrt(var + 1e-3)
    z = (y2_ref[0] - mean) * inv * g_ref[...] + b_ref[...]
    out_ref[0] = jnp.where(z >= 0, z, 0.01 * z)


def kernel(xyz, skip, xyz_prev, feat_prev, W1, g1, b1, W2, g2, b2):
    B, _, N0 = xyz.shape
    N = xyz_prev.shape[2]
    Cprev = feat_prev.shape[1]
    Cskip = skip.shape[1]
    C1 = W1.shape[0]
    C2 = W2.shape[0]
    TQ = 512
    nt = N0 // TQ
    count = float(B * N0)

    p_t = jnp.transpose(xyz_prev, (0, 2, 1))

    ind_t, w_t = pl.pallas_call(
        functools.partial(_three_nn_kernel, n_keys=N),
        grid=(B, nt),
        in_specs=[pl.BlockSpec((1, 3, TQ), lambda b, t: (b, 0, t)),
                  pl.BlockSpec((1, N, 3), lambda b, t: (b, 0, 0))],
        out_specs=[pl.BlockSpec((1, 3, TQ), lambda b, t: (b, 0, t)),
                   pl.BlockSpec((1, 3, TQ), lambda b, t: (b, 0, t))],
        out_shape=[jax.ShapeDtypeStruct((B, 3, N0), jnp.int32),
                   jax.ShapeDtypeStruct((B, 3, N0), jnp.float32)],
        interpret=_INTERPRET,
    )(xyz, p_t)

    y1, st1 = pl.pallas_call(
        functools.partial(_interp_conv1_kernel, n_keys=N, cprev=Cprev),
        grid=(B, nt),
        in_specs=[pl.BlockSpec((1, 3, TQ), lambda b, t: (b, 0, t)),
                  pl.BlockSpec((1, 3, TQ), lambda b, t: (b, 0, t)),
                  pl.BlockSpec((1, Cprev, N), lambda b, t: (b, 0, 0)),
                  pl.BlockSpec((1, Cskip, TQ), lambda b, t: (b, 0, t)),
                  pl.BlockSpec((C1, Cprev + Cskip), lambda b, t: (0, 0))],
        out_specs=[pl.BlockSpec((1, C1, TQ), lambda b, t: (b, 0, t)),
                   pl.BlockSpec((C1, 2), lambda b, t: (0, 0))],
        out_shape=[jax.ShapeDtypeStruct((B, C1, N0), jnp.float32),
                   jax.ShapeDtypeStruct((C1, 2), jnp.float32)],
        interpret=_INTERPRET,
    )(ind_t, w_t, feat_prev, skip, W1)

    y2, st2 = pl.pallas_call(
        functools.partial(_bn_conv2_kernel, count=count),
        grid=(B, nt),
        in_specs=[pl.BlockSpec((1, C1, TQ), lambda b, t: (b, 0, t)),
                  pl.BlockSpec((C1, 2), lambda b, t: (0, 0)),
                  pl.BlockSpec((C1, 1), lambda b, t: (0, 0)),
                  pl.BlockSpec((C1, 1), lambda b, t: (0, 0)),
                  pl.BlockSpec((C2, C1), lambda b, t: (0, 0))],
        out_specs=[pl.BlockSpec((1, C2, TQ), lambda b, t: (b, 0, t)),
                   pl.BlockSpec((C2, 2), lambda b, t: (0, 0))],
        out_shape=[jax.ShapeDtypeStruct((B, C2, N0), jnp.float32),
                   jax.ShapeDtypeStruct((C2, 2), jnp.float32)],
        interpret=_INTERPRET,
    )(y1, st1, g1.reshape(-1, 1), b1.reshape(-1, 1), W2)

    y = pl.pallas_call(
        functools.partial(_bn_out_kernel, count=count),
        grid=(B, nt),
        in_specs=[pl.BlockSpec((1, C2, TQ), lambda b, t: (b, 0, t)),
                  pl.BlockSpec((C2, 2), lambda b, t: (0, 0)),
                  pl.BlockSpec((C2, 1), lambda b, t: (0, 0)),
                  pl.BlockSpec((C2, 1), lambda b, t: (0, 0))],
        out_specs=pl.BlockSpec((1, C2, TQ), lambda b, t: (b, 0, t)),
        out_shape=jax.ShapeDtypeStruct((B, C2, N0), jnp.float32),
        interpret=_INTERPRET,
    )(y2, st2, g2.reshape(-1, 1), b2.reshape(-1, 1))

    return (xyz, y)
